# baseline (speedup 1.0000x reference)
import jax
import jax.numpy as jnp
from jax import lax
from jax.experimental import pallas as pl
from jax.experimental.pallas import tpu as pltpu

_H, _W, _F = 100, 100, 128
_K = _H * _W
_KPAD = 10240
_BB = 1024
_KB = 2048
_NK = _KPAD // _KB
_NSUB = _KB // 128
_BIG = 2 ** 30


def _som_kernel(x_ref, w_ref, out_ref, b2h_ref):
    i = pl.program_id(0)

    @pl.when(i == 0)
    def _build_b2half():
        for k in range(_NK):
            wc = w_ref[k * _KB:(k + 1) * _KB, :]
            b2h = 0.5 * jnp.sum(wc * wc, axis=1)
            b2h_ref[0:1, k * _KB:(k + 1) * _KB] = b2h[None, :]

    x = x_ref[:]
    lane = lax.broadcasted_iota(jnp.int32, (_BB, 128), 1)
    run_min = jnp.full((_BB, 128), jnp.inf, dtype=jnp.float32)
    run_sub = jnp.zeros((_BB, 128), dtype=jnp.int32)
    tail = _K - (_KPAD - _KB)
    for k in range(_NK):
        wc = w_ref[k * _KB:(k + 1) * _KB, :]
        dot = lax.dot_general(x, wc, (((1,), (1,)), ((), ())),
                              preferred_element_type=jnp.float32)
        e = b2h_ref[0:1, k * _KB:(k + 1) * _KB] - dot
        for j in range(_NSUB):
            et = e[:, j * 128:(j + 1) * 128]
            if k == _NK - 1 and (j + 1) * 128 > tail:
                gcol = k * _KB + j * 128 + lane
                et = jnp.where(gcol < _K, et, jnp.inf)
            cmp = et < run_min
            run_min = jnp.minimum(run_min, et)
            run_sub = jnp.where(cmp, jnp.int32(k * _NSUB + j), run_sub)

    gm = jnp.min(run_min, axis=1, keepdims=True)
    gidx = run_sub * 128 + lane
    idx = jnp.min(jnp.where(run_min == gm, gidx, _BIG),
                  axis=1, keepdims=True)
    out_ref[:] = jnp.concatenate([idx // _W, idx % _W], axis=1)


@jax.jit
def kernel(xb, weights):
    n = xb.shape[0]
    w_flat = weights.reshape(_K, _F)
    grid = (n // _BB,)
    return pl.pallas_call(
        _som_kernel,
        grid=grid,
        in_specs=[
            pl.BlockSpec((_BB, _F), lambda i: (i, 0)),
            pl.BlockSpec((_KPAD, _F), lambda i: (0, 0)),
        ],
        out_specs=pl.BlockSpec((_BB, 2), lambda i: (i, 0)),
        out_shape=jax.ShapeDtypeStruct((n, 2), jnp.int32),
        scratch_shapes=[
            pltpu.VMEM((8, _KPAD), jnp.float32),
        ],
        compiler_params=pltpu.CompilerParams(
            dimension_semantics=("arbitrary",),
        ),
    )(xb, w_flat)


# speedup vs baseline: 1.0153x; 1.0153x over previous
import jax
import jax.numpy as jnp
from jax import lax
from jax.experimental import pallas as pl
from jax.experimental.pallas import tpu as pltpu

_H, _W, _F = 100, 100, 128
_K = _H * _W
_KPAD = 10240
_BB = 2048
_KB = 2048
_NK = _KPAD // _KB
_NSUB = _KB // 128
_BIG = 2 ** 30


def _som_kernel(x_ref, w_ref, out_ref, b2h_ref):
    i = pl.program_id(0)

    @pl.when(i == 0)
    def _build_b2half():
        for k in range(_NK):
            wc = w_ref[k * _KB:(k + 1) * _KB, :]
            b2h = 0.5 * jnp.sum(wc * wc, axis=1)
            b2h_ref[0:1, k * _KB:(k + 1) * _KB] = b2h[None, :]

    x = x_ref[:]
    lane = lax.broadcasted_iota(jnp.int32, (_BB, 128), 1)
    run_min = jnp.full((_BB, 128), jnp.inf, dtype=jnp.float32)
    run_sub = jnp.zeros((_BB, 128), dtype=jnp.int32)
    tail = _K - (_KPAD - _KB)
    for k in range(_NK):
        wc = w_ref[k * _KB:(k + 1) * _KB, :]
        dot = lax.dot_general(x, wc, (((1,), (1,)), ((), ())),
                              preferred_element_type=jnp.float32)
        e = b2h_ref[0:1, k * _KB:(k + 1) * _KB] - dot
        for j in range(_NSUB):
            et = e[:, j * 128:(j + 1) * 128]
            if k == _NK - 1 and (j + 1) * 128 > tail:
                gcol = k * _KB + j * 128 + lane
                et = jnp.where(gcol < _K, et, jnp.inf)
            cmp = et < run_min
            run_min = jnp.minimum(run_min, et)
            run_sub = jnp.where(cmp, jnp.int32(k * _NSUB + j), run_sub)

    gm = jnp.min(run_min, axis=1, keepdims=True)
    gidx = run_sub * 128 + lane
    idx = jnp.min(jnp.where(run_min == gm, gidx, _BIG),
                  axis=1, keepdims=True)
    out_ref[:] = jnp.concatenate([idx // _W, idx % _W], axis=1)


@jax.jit
def kernel(xb, weights):
    n = xb.shape[0]
    w_flat = weights.reshape(_K, _F)
    grid = (n // _BB,)
    return pl.pallas_call(
        _som_kernel,
        grid=grid,
        in_specs=[
            pl.BlockSpec((_BB, _F), lambda i: (i, 0)),
            pl.BlockSpec((_KPAD, _F), lambda i: (0, 0)),
        ],
        out_specs=pl.BlockSpec((_BB, 2), lambda i: (i, 0)),
        out_shape=jax.ShapeDtypeStruct((n, 2), jnp.int32),
        scratch_shapes=[
            pltpu.VMEM((8, _KPAD), jnp.float32),
        ],
        compiler_params=pltpu.CompilerParams(
            dimension_semantics=("arbitrary",),
        ),
    )(xb, w_flat)
